# Initial kernel scaffold; baseline (speedup 1.0000x reference)
import jax
import jax.numpy as jnp
from jax.experimental import pallas as pl


def kernel(node_type, edge_index, distance, params):
    raise NotImplementedError("write your pallas kernel here")


# speedup vs baseline: 4.5110x; 4.5110x over previous
import functools

import numpy as np
import jax
import jax.numpy as jnp
from jax import lax
from jax.experimental import pallas as pl
from jax.experimental.pallas import tpu as pltpu
from jax.experimental.pallas import tpu_sc as plsc

_N = 10000
_E = 320000
_D = 128
_TYPES = 100
_ETAB = 3000
_RBF_C = np.linspace(0.0, 5.0, 5).astype(np.float32)
_RBF_INVGAP = 1.0 / float(_RBF_C[1] - _RBF_C[0])

_K = 80
_NBLK = _E // _K
_NW = 32
_BPW = _NBLK // _NW
_NP = 10240
_RPT = _NP // 16

_BN = 2000
_GN = _N // _BN
_BE = 2000
_GE = _E // _BE


def _sp05(x):
    h = 0.5 * x
    return 2.0 * (jnp.maximum(h, 0.0) + jnp.log(1.0 + jnp.exp(-jnp.abs(h))))


def _sp1(x):
    return jnp.maximum(x, 0.0) + jnp.log(1.0 + jnp.exp(-jnp.abs(x)))


def _prep_body(nt_ref, atom_ref, eemb_ref,
               v0w, v0b, e0w, e0b, v1w, v1b, e1w, e1b, v2w, v2b, e2w, e2b,
               node0_ref, tp0_ref, tp1_ref, tp2_ref):
    nt = nt_ref[:]
    oh = (nt == lax.broadcasted_iota(jnp.int32, (_N, _TYPES), 1)).astype(jnp.float32)
    node0_ref[:] = jnp.dot(oh, atom_ref[:], preferred_element_type=jnp.float32)
    t = eemb_ref[:]
    for vw, vb, ew, eb, tp_ref in ((v0w, v0b, e0w, e0b, tp0_ref),
                                   (v1w, v1b, e1w, e1b, tp1_ref),
                                   (v2w, v2b, e2w, e2b, tp2_ref)):
        tp = jnp.dot(t, vw[:], preferred_element_type=jnp.float32) + vb[:]
        tp_ref[:] = tp
        t = _sp05(jnp.dot(tp, ew[:], preferred_element_type=jnp.float32) + eb[:])


def _h_body(d_ref,
            w10, b10, w20, b20, w11, b11, w21, b21, w12, b12, w22, b22,
            h0_ref, h1_ref, h2_ref):
    d = d_ref[:]
    gap = 1.0 / _RBF_INVGAP
    c = lax.broadcasted_iota(jnp.int32, (1, _RBF_C.shape[0]), 1).astype(jnp.float32) * gap
    rbf = jnp.exp(-_RBF_INVGAP * (d - c) ** 2)
    for w1, b1, w2, b2, h_ref in ((w10, b10, w20, b20, h0_ref),
                                  (w11, b11, w21, b21, h1_ref),
                                  (w12, b12, w22, b22, h2_ref)):
        s = _sp05(jnp.dot(rbf, w1[:], preferred_element_type=jnp.float32) + b1[:])
        h_ref[:] = jnp.dot(s, w2[:], preferred_element_type=jnp.float32) + b2[:]


def _nn_body(node_ref, w_ref, b_ref, out_ref):
    out_ref[:] = jnp.dot(node_ref[:], w_ref[:],
                         preferred_element_type=jnp.float32) + b_ref[:]


def _upd_body(parts_ref, node_ref, w2, b2, w3, b3, out_ref):
    agg = parts_ref[0] + parts_ref[1]
    x = _sp05(jnp.dot(agg, w2[:], preferred_element_type=jnp.float32) + b2[:])
    out_ref[:] = node_ref[:] + jnp.dot(x, w3[:],
                                       preferred_element_type=jnp.float32) + b3[:]


def _ro_body(n0, n1, n2, n3, w0, w1, w2, w3, b1, d2w, d2b, out_ref):
    i = pl.program_id(0)
    h = (jnp.dot(n0[:], w0[:], preferred_element_type=jnp.float32)
         + jnp.dot(n1[:], w1[:], preferred_element_type=jnp.float32)
         + jnp.dot(n2[:], w2[:], preferred_element_type=jnp.float32)
         + jnp.dot(n3[:], w3[:], preferred_element_type=jnp.float32)
         + b1[:])
    h = _sp1(h)
    r = jnp.dot(h, d2w[:], preferred_element_type=jnp.float32)
    blocksum = jnp.sum(r) + _BN * d2b[0, 0]

    @pl.when(i == 0)
    def _():
        out_ref[:] = jnp.zeros((1, 1), jnp.float32)

    out_ref[:] = out_ref[:] + blocksum.reshape(1, 1)


def _etype_body(sd_hbm, nt_hbm, et_hbm, sd_v, tx_v, ty_v, et_v, sem):
    wid = lax.axis_index("c") * 16 + lax.axis_index("s")

    def blk(j, carry):
        b = wid * _BPW + j
        pltpu.sync_copy(sd_hbm.at[b], sd_v)
        pltpu.async_copy(nt_hbm.at[sd_v.at[0]], tx_v, sem).wait()
        pltpu.async_copy(nt_hbm.at[sd_v.at[1]], ty_v, sem).wait()
        for cc in range(_K // 16):
            sl = pl.ds(cc * 16, 16)
            tx = tx_v[sl]
            ty = ty_v[sl]
            k = jnp.abs(tx - ty) - 1
            et_v[sl] = tx * ty + ((k * k) >> 2)
        pltpu.sync_copy(et_v, et_hbm.at[pl.ds(b * _K, _K)])
        return carry

    lax.fori_loop(0, _BPW, blk, 0)


def _agg_body(eidx, h, nn, tp, out,
              idx_v, h_v, nn_v, t_v, acc, sem1, sem2):
    c = lax.axis_index("c")
    s = lax.axis_index("s")
    wid = c * 16 + s

    def zrow(r, carry):
        for cc in range(_D // 16):
            h_v[r, pl.ds(cc * 16, 16)] = jnp.zeros((16,), jnp.float32)
        return carry

    lax.fori_loop(0, _K, zrow, 0)
    r0 = s * _RPT
    for kk in range(_RPT // _K):
        pltpu.sync_copy(h_v, acc.at[pl.ds(r0 + kk * _K, _K)])
    plsc.subcore_barrier()

    def blk(j, carry):
        b = wid * _BPW + j
        pltpu.sync_copy(eidx.at[b], idx_v)
        cp1 = pltpu.async_copy(nn.at[idx_v.at[0]], nn_v, sem1)
        cp2 = pltpu.async_copy(tp.at[idx_v.at[1]], t_v, sem2)
        pltpu.sync_copy(h.at[pl.ds(b * _K, _K)], h_v)
        cp1.wait()
        cp2.wait()

        def fma(r, carry2):
            for cc in range(_D // 16):
                sl = pl.ds(cc * 16, 16)
                h_v[r, sl] = nn_v[r, sl] * h_v[r, sl] + t_v[r, sl]
            return carry2

        lax.fori_loop(0, _K, fma, 0)
        pltpu.sync_copy(h_v, acc.at[idx_v.at[2]], add=True)
        return carry

    lax.fori_loop(0, _BPW, blk, 0)
    plsc.subcore_barrier()
    pltpu.sync_copy(acc.at[pl.ds(r0, _RPT)], out.at[c, pl.ds(r0, _RPT)])


@functools.lru_cache(maxsize=None)
def _etype_kernel_build():
    mesh = plsc.VectorSubcoreMesh(core_axis_name="c", subcore_axis_name="s")
    return pl.kernel(
        _etype_body,
        out_type=jax.ShapeDtypeStruct((_E,), jnp.int32),
        mesh=mesh,
        scratch_types=[
            pltpu.VMEM((2, _K), jnp.int32),
            pltpu.VMEM((_K,), jnp.int32),
            pltpu.VMEM((_K,), jnp.int32),
            pltpu.VMEM((_K,), jnp.int32),
            pltpu.SemaphoreType.DMA,
        ],
    )


def _etype_call(sd, nt):
    return _etype_kernel_build()(sd, nt)


@functools.lru_cache(maxsize=None)
def _agg_kernel_build():
    mesh = plsc.VectorSubcoreMesh(core_axis_name="c", subcore_axis_name="s")
    return pl.kernel(
        _agg_body,
        out_type=jax.ShapeDtypeStruct((2, _NP, _D), jnp.float32),
        mesh=mesh,
        scratch_types=[
            pltpu.VMEM((3, _K), jnp.int32),
            pltpu.VMEM((_K, _D), jnp.float32),
            pltpu.VMEM((_K, _D), jnp.float32),
            pltpu.VMEM((_K, _D), jnp.float32),
            pltpu.VMEM_SHARED((_NP, _D), jnp.float32),
            pltpu.SemaphoreType.DMA,
            pltpu.SemaphoreType.DMA,
        ],
    )


def _agg_call(eidx, h, nn, tp):
    return _agg_kernel_build()(eidx, h, nn, tp)


def _prep_call(nt2, atom, eemb, wlist):
    return pl.pallas_call(
        _prep_body,
        out_shape=[
            jax.ShapeDtypeStruct((_N, _D), jnp.float32),
            jax.ShapeDtypeStruct((_ETAB, _D), jnp.float32),
            jax.ShapeDtypeStruct((_ETAB, _D), jnp.float32),
            jax.ShapeDtypeStruct((_ETAB, _D), jnp.float32),
        ],
    )(nt2, atom, eemb, *wlist)


def _h_call(dist2, wlist):
    bcast = lambda shape: pl.BlockSpec(shape, lambda i: (0, 0))
    wspecs = []
    for w in wlist:
        wspecs.append(bcast(w.shape))
    return pl.pallas_call(
        _h_body,
        grid=(_GE,),
        in_specs=[pl.BlockSpec((_BE, 1), lambda i: (i, 0))] + wspecs,
        out_specs=[pl.BlockSpec((_BE, _D), lambda i: (i, 0))] * 3,
        out_shape=[jax.ShapeDtypeStruct((_E, _D), jnp.float32)] * 3,
    )(dist2, *wlist)


def _nn_call(node, w, b):
    return pl.pallas_call(
        _nn_body,
        grid=(_GN,),
        in_specs=[
            pl.BlockSpec((_BN, _D), lambda i: (i, 0)),
            pl.BlockSpec((_D, _D), lambda i: (0, 0)),
            pl.BlockSpec((1, _D), lambda i: (0, 0)),
        ],
        out_specs=pl.BlockSpec((_BN, _D), lambda i: (i, 0)),
        out_shape=jax.ShapeDtypeStruct((_N, _D), jnp.float32),
    )(node, w, b)


def _upd_call(parts, node, w2, b2, w3, b3):
    return pl.pallas_call(
        _upd_body,
        grid=(_GN,),
        in_specs=[
            pl.BlockSpec((2, _BN, _D), lambda i: (0, i, 0)),
            pl.BlockSpec((_BN, _D), lambda i: (i, 0)),
            pl.BlockSpec((_D, _D), lambda i: (0, 0)),
            pl.BlockSpec((1, _D), lambda i: (0, 0)),
            pl.BlockSpec((_D, _D), lambda i: (0, 0)),
            pl.BlockSpec((1, _D), lambda i: (0, 0)),
        ],
        out_specs=pl.BlockSpec((_BN, _D), lambda i: (i, 0)),
        out_shape=jax.ShapeDtypeStruct((_N, _D), jnp.float32),
    )(parts, node, w2, b2, w3, b3)


def _ro_call(nodes, d1ws, d1b, d2w, d2b):
    nspec = pl.BlockSpec((_BN, _D), lambda i: (i, 0))
    wspec = pl.BlockSpec((_D, 64), lambda i: (0, 0))
    return pl.pallas_call(
        _ro_body,
        grid=(_GN,),
        in_specs=[nspec] * 4 + [wspec] * 4 + [
            pl.BlockSpec((1, 64), lambda i: (0, 0)),
            pl.BlockSpec((64, 1), lambda i: (0, 0)),
            pl.BlockSpec((1, 1), lambda i: (0, 0)),
        ],
        out_specs=pl.BlockSpec((1, 1), lambda i: (0, 0)),
        out_shape=jax.ShapeDtypeStruct((1, 1), jnp.float32),
    )(*nodes, *d1ws, d1b, d2w, d2b)


def kernel(node_type, edge_index, distance, params):
    p = params
    nt = node_type.astype(jnp.int32)
    src = edge_index[0].astype(jnp.int32)
    dst = edge_index[1].astype(jnp.int32)
    dist2 = distance.astype(jnp.float32).reshape(_E, 1)
    convs = [p['conv_%d' % i] for i in range(3)]
    rb = lambda x: x.reshape(1, -1)

    prep_w = []
    for cv in convs:
        prep_w += [cv['ve3_w'], rb(cv['ve3_b']), cv['el1_w'], rb(cv['el1_b'])]
    node0, tp0, tp1, tp2 = _prep_call(nt.reshape(_N, 1), p['atom_emb'],
                                      p['edge_emb'], prep_w)

    h_w = []
    for cv in convs:
        h_w += [cv['ve1_w'], rb(cv['ve1_b']), cv['ve2_w'], rb(cv['ve2_b'])]
    hs = _h_call(dist2, h_w)

    sd = jnp.stack([src.reshape(_NBLK, _K), dst.reshape(_NBLK, _K)], axis=1)
    etype = _etype_call(sd, nt)
    eidx = jnp.stack([src.reshape(_NBLK, _K),
                      etype.reshape(_NBLK, _K),
                      dst.reshape(_NBLK, _K)], axis=1)

    tps = [tp0, tp1, tp2]
    node = node0
    nodes = [node0]
    for i in range(3):
        cv = convs[i]
        nn = _nn_call(node, cv['nl1_w'], rb(cv['nl1_b']))
        parts = _agg_call(eidx, hs[i], nn, tps[i])
        node = _upd_call(parts, node, cv['nl2_w'], rb(cv['nl2_b']),
                         cv['nl3_w'], rb(cv['nl3_b']))
        nodes.append(node)

    d1ws = [p['d1_w'][i * _D:(i + 1) * _D] for i in range(4)]
    return _ro_call(nodes, d1ws, rb(p['d1_b']), p['d2_w'],
                    p['d2_b'].reshape(1, 1))


# speedup vs baseline: 5.9580x; 1.3208x over previous
import functools

import numpy as np
import jax
import jax.numpy as jnp
from jax import lax
from jax.experimental import pallas as pl
from jax.experimental.pallas import tpu as pltpu
from jax.experimental.pallas import tpu_sc as plsc

_N = 10000
_E = 320000
_D = 128
_TYPES = 100
_ETAB = 3000
_RBF_C = np.linspace(0.0, 5.0, 5).astype(np.float32)
_RBF_INVGAP = 1.0 / float(_RBF_C[1] - _RBF_C[0])

_NW = 32
_EPT = _E // _NW
_KA = 56
_NBA = _EPT // _KA
_TA = _EPT - _NBA * _KA
_KE = 128
_NBE = _EPT // _KE
_TE = _EPT - _NBE * _KE

_BN = 2000
_GN = _N // _BN
_BE = 2000
_GE = _E // _BE


def _sp05(x):
    h = 0.5 * x
    return 2.0 * (jnp.maximum(h, 0.0) + jnp.log(1.0 + jnp.exp(-jnp.abs(h))))


def _sp1(x):
    return jnp.maximum(x, 0.0) + jnp.log(1.0 + jnp.exp(-jnp.abs(x)))


def _prep_body(nt_ref, atom_ref, eemb_ref,
               v0w, v0b, e0w, e0b, v1w, v1b, e1w, e1b, v2w, v2b, e2w, e2b,
               node0_ref, tp0_ref, tp1_ref, tp2_ref):
    nt = nt_ref[:]
    oh = (nt == lax.broadcasted_iota(jnp.int32, (_N, _TYPES), 1)).astype(jnp.float32)
    node0_ref[:] = jnp.dot(oh, atom_ref[:], preferred_element_type=jnp.float32)
    t = eemb_ref[:]
    for vw, vb, ew, eb, tp_ref in ((v0w, v0b, e0w, e0b, tp0_ref),
                                   (v1w, v1b, e1w, e1b, tp1_ref),
                                   (v2w, v2b, e2w, e2b, tp2_ref)):
        tp = jnp.dot(t, vw[:], preferred_element_type=jnp.float32) + vb[:]
        tp_ref[:] = tp
        t = _sp05(jnp.dot(tp, ew[:], preferred_element_type=jnp.float32) + eb[:])


def _h_body(d_ref,
            w10, b10, w20, b20, w11, b11, w21, b21, w12, b12, w22, b22,
            h0_ref, h1_ref, h2_ref):
    d = d_ref[:]
    gap = 1.0 / _RBF_INVGAP
    c = lax.broadcasted_iota(jnp.int32, (1, _RBF_C.shape[0]), 1).astype(jnp.float32) * gap
    rbf = jnp.exp(-_RBF_INVGAP * (d - c) ** 2)
    for w1, b1, w2, b2, h_ref in ((w10, b10, w20, b20, h0_ref),
                                  (w11, b11, w21, b21, h1_ref),
                                  (w12, b12, w22, b22, h2_ref)):
        s = _sp05(jnp.dot(rbf, w1[:], preferred_element_type=jnp.float32) + b1[:])
        h_ref[:] = jnp.dot(s, w2[:], preferred_element_type=jnp.float32) + b2[:]


def _nn_body(node_ref, w_ref, b_ref, out_ref):
    out_ref[:] = jnp.dot(node_ref[:], w_ref[:],
                         preferred_element_type=jnp.float32) + b_ref[:]


def _upd_body(parts_ref, node_ref, w2, b2, w3, b3, out_ref):
    agg = parts_ref[0] + parts_ref[1]
    x = _sp05(jnp.dot(agg, w2[:], preferred_element_type=jnp.float32) + b2[:])
    out_ref[:] = node_ref[:] + jnp.dot(x, w3[:],
                                       preferred_element_type=jnp.float32) + b3[:]


def _ro_body(n0, n1, n2, n3, w0, w1, w2, w3, b1, d2w, d2b, out_ref):
    i = pl.program_id(0)
    h = (jnp.dot(n0[:], w0[:], preferred_element_type=jnp.float32)
         + jnp.dot(n1[:], w1[:], preferred_element_type=jnp.float32)
         + jnp.dot(n2[:], w2[:], preferred_element_type=jnp.float32)
         + jnp.dot(n3[:], w3[:], preferred_element_type=jnp.float32)
         + b1[:])
    h = _sp1(h)
    r = jnp.dot(h, d2w[:], preferred_element_type=jnp.float32)
    blocksum = jnp.sum(r) + _BN * d2b[0, 0]

    @pl.when(i == 0)
    def _():
        out_ref[:] = jnp.zeros((1, 1), jnp.float32)

    out_ref[:] = out_ref[:] + blocksum.reshape(1, 1)


def _etype_body(srch, dsth, nt, et_hbm,
                sd_v, tx0, tx1, ty0, ty1, et0_v, et1_v, tsd,
                ssd0, ssd1, stx0, stx1, sty0, sty1, so0, so1):
    c = lax.axis_index("c")
    s = lax.axis_index("s")
    wid = c * 16 + s
    ebase = wid * _EPT
    tx_v = (tx0, tx1)
    ty_v = (ty0, ty1)
    et_v = (et0_v, et1_v)
    sem_sd = (ssd0, ssd1)
    sem_tx = (stx0, stx1)
    sem_ty = (sty0, sty1)
    sem_out = (so0, so1)

    for p in (0, 1):
        e0 = ebase + p * _KE
        pltpu.sync_copy(srch.at[pl.ds(e0, _KE)], sd_v.at[p, 0])
        pltpu.sync_copy(dsth.at[pl.ds(e0, _KE)], sd_v.at[p, 1])
        pltpu.async_copy(nt.at[sd_v.at[p, 0]], tx_v[p], sem_tx[p])
        pltpu.async_copy(nt.at[sd_v.at[p, 1]], ty_v[p], sem_ty[p])

    def it(j2, cy):
        for p in (0, 1):
            b = 2 * j2 + p
            e0 = ebase + b * _KE
            e2 = e0 + 2 * _KE
            q_cur = b % 4
            q_nxt = (b + 2) % 4

            @pl.when(b + 2 < _NBE)
            def _():
                pltpu.async_copy(srch.at[pl.ds(e2, _KE)], sd_v.at[q_nxt, 0],
                                 sem_sd[p])
                pltpu.async_copy(dsth.at[pl.ds(e2, _KE)], sd_v.at[q_nxt, 1],
                                 sem_sd[p])

            pltpu.make_async_copy(nt.at[sd_v.at[q_cur, 0]], tx_v[p],
                                  sem_tx[p]).wait()
            pltpu.make_async_copy(nt.at[sd_v.at[q_cur, 1]], ty_v[p],
                                  sem_ty[p]).wait()

            @pl.when(b >= 2)
            def _():
                pltpu.make_async_copy(et_v[p], et_hbm.at[pl.ds(e0, _KE)],
                                      sem_out[p]).wait()

            for cc in range(_KE // 16):
                sl = pl.ds(cc * 16, 16)
                tx = tx_v[p][sl]
                ty = ty_v[p][sl]
                k = jnp.abs(tx - ty) - 1
                et_v[p][sl] = tx * ty + ((k * k) >> 2)
            pltpu.async_copy(et_v[p], et_hbm.at[pl.ds(e0, _KE)], sem_out[p])

            @pl.when(b + 2 < _NBE)
            def _():
                pltpu.make_async_copy(srch.at[pl.ds(e2, _KE)],
                                      sd_v.at[q_nxt, 0], sem_sd[p]).wait()
                pltpu.make_async_copy(dsth.at[pl.ds(e2, _KE)],
                                      sd_v.at[q_nxt, 1], sem_sd[p]).wait()
                pltpu.async_copy(nt.at[sd_v.at[q_nxt, 0]], tx_v[p], sem_tx[p])
                pltpu.async_copy(nt.at[sd_v.at[q_nxt, 1]], ty_v[p], sem_ty[p])
        return cy

    lax.fori_loop(0, _NBE // 2, it, 0)
    for p in (0, 1):
        pltpu.make_async_copy(et_v[p], et_hbm.at[pl.ds(ebase, _KE)],
                              sem_out[p]).wait()

    et0 = ebase + _NBE * _KE
    tsl = pl.ds(0, _TE)
    pltpu.sync_copy(srch.at[pl.ds(et0, _TE)], tsd.at[0])
    pltpu.sync_copy(dsth.at[pl.ds(et0, _TE)], tsd.at[1])
    pltpu.async_copy(nt.at[tsd.at[0]], tx_v[0].at[tsl], sem_tx[0]).wait()
    pltpu.async_copy(nt.at[tsd.at[1]], ty_v[0].at[tsl], sem_ty[0]).wait()
    tx = tx_v[0][tsl]
    ty = ty_v[0][tsl]
    k = jnp.abs(tx - ty) - 1
    et_v[0][tsl] = tx * ty + ((k * k) >> 2)
    pltpu.sync_copy(et_v[0].at[tsl], et_hbm.at[pl.ds(et0, _TE)])


def _agg_body(srch, eth, dsth, h, nn, tp, out,
              idx_v, hv0, hv1, nv0, nv1, tv0, tv1, tidx, acc,
              si0, si1, sh0, sh1, sn0, sn1, st0, st1, sc0, sc1):
    c = lax.axis_index("c")
    s = lax.axis_index("s")
    wid = c * 16 + s
    ebase = wid * _EPT
    h_v = (hv0, hv1)
    nn_v = (nv0, nv1)
    t_v = (tv0, tv1)
    sem_idx = (si0, si1)
    sem_h = (sh0, sh1)
    sem_nn = (sn0, sn1)
    sem_tp = (st0, st1)
    sem_sc = (sc0, sc1)

    def zrow(r, cy):
        for cc in range(_D // 16):
            hv0[r, pl.ds(cc * 16, 16)] = jnp.zeros((16,), jnp.float32)
        return cy

    lax.fori_loop(0, _KA, zrow, 0)
    r0 = s * 624

    @pl.when(s < 15)
    def _():
        for kk in range(11):
            pltpu.sync_copy(hv0, acc.at[pl.ds(r0 + kk * _KA, _KA)])
        pltpu.sync_copy(hv0.at[pl.ds(0, 8)], acc.at[pl.ds(r0 + 616, 8)])

    @pl.when(s == 15)
    def _():
        for kk in range(11):
            pltpu.sync_copy(hv0, acc.at[pl.ds(9360 + kk * _KA, _KA)])
        pltpu.sync_copy(hv0.at[pl.ds(0, 24)], acc.at[pl.ds(9976, 24)])

    for p in (0, 1):
        e0 = ebase + p * _KA
        pltpu.sync_copy(srch.at[pl.ds(e0, _KA)], idx_v.at[p, 0])
        pltpu.sync_copy(eth.at[pl.ds(e0, _KA)], idx_v.at[p, 1])
        pltpu.sync_copy(dsth.at[pl.ds(e0, _KA)], idx_v.at[p, 2])
        pltpu.async_copy(h.at[pl.ds(e0, _KA)], h_v[p], sem_h[p])
        pltpu.async_copy(nn.at[idx_v.at[p, 0]], nn_v[p], sem_nn[p])
        pltpu.async_copy(tp.at[idx_v.at[p, 1]], t_v[p], sem_tp[p])
    plsc.subcore_barrier()

    def it(j2, cy):
        for p in (0, 1):
            b = 2 * j2 + p
            e0 = ebase + b * _KA
            e2 = e0 + 2 * _KA
            q_cur = b % 4
            q_nxt = (b + 2) % 4

            @pl.when(b + 2 < _NBA)
            def _():
                pltpu.async_copy(srch.at[pl.ds(e2, _KA)], idx_v.at[q_nxt, 0],
                                 sem_idx[p])
                pltpu.async_copy(eth.at[pl.ds(e2, _KA)], idx_v.at[q_nxt, 1],
                                 sem_idx[p])
                pltpu.async_copy(dsth.at[pl.ds(e2, _KA)], idx_v.at[q_nxt, 2],
                                 sem_idx[p])

            pltpu.make_async_copy(h.at[pl.ds(e0, _KA)], h_v[p],
                                  sem_h[p]).wait()
            pltpu.make_async_copy(nn.at[idx_v.at[q_cur, 0]], nn_v[p],
                                  sem_nn[p]).wait()
            pltpu.make_async_copy(tp.at[idx_v.at[q_cur, 1]], t_v[p],
                                  sem_tp[p]).wait()

            hv, nv, tv = h_v[p], nn_v[p], t_v[p]

            def fma(r, c2):
                for cc in range(_D // 16):
                    sl = pl.ds(cc * 16, 16)
                    hv[r, sl] = nv[r, sl] * hv[r, sl] + tv[r, sl]
                return c2

            lax.fori_loop(0, _KA, fma, 0)
            pltpu.async_copy(hv, acc.at[idx_v.at[q_cur, 2]], sem_sc[p],
                             add=True)

            @pl.when(b + 2 < _NBA)
            def _():
                pltpu.make_async_copy(srch.at[pl.ds(e2, _KA)],
                                      idx_v.at[q_nxt, 0], sem_idx[p]).wait()
                pltpu.make_async_copy(eth.at[pl.ds(e2, _KA)],
                                      idx_v.at[q_nxt, 1], sem_idx[p]).wait()
                pltpu.make_async_copy(dsth.at[pl.ds(e2, _KA)],
                                      idx_v.at[q_nxt, 2], sem_idx[p]).wait()
                pltpu.async_copy(nn.at[idx_v.at[q_nxt, 0]], nn_v[p], sem_nn[p])
                pltpu.async_copy(tp.at[idx_v.at[q_nxt, 1]], t_v[p], sem_tp[p])
                pltpu.make_async_copy(hv, acc.at[idx_v.at[q_cur, 2]],
                                      sem_sc[p]).wait()
                pltpu.async_copy(h.at[pl.ds(e2, _KA)], hv, sem_h[p])
        return cy

    lax.fori_loop(0, _NBA // 2, it, 0)
    for p in (0, 1):
        pltpu.make_async_copy(h_v[p], acc.at[idx_v.at[0, 2]],
                              sem_sc[p]).wait()

    et0 = ebase + _NBA * _KA
    tsl = pl.ds(0, _TA)
    pltpu.sync_copy(srch.at[pl.ds(et0, _TA)], tidx.at[0])
    pltpu.sync_copy(eth.at[pl.ds(et0, _TA)], tidx.at[1])
    pltpu.sync_copy(dsth.at[pl.ds(et0, _TA)], tidx.at[2])
    pltpu.async_copy(nn.at[tidx.at[0]], nv0.at[tsl], sem_nn[0]).wait()
    pltpu.async_copy(tp.at[tidx.at[1]], tv0.at[tsl], sem_tp[0]).wait()
    pltpu.sync_copy(h.at[pl.ds(et0, _TA)], hv0.at[tsl])

    def tfma(r, c2):
        for cc in range(_D // 16):
            sl = pl.ds(cc * 16, 16)
            hv0[r, sl] = nv0[r, sl] * hv0[r, sl] + tv0[r, sl]
        return c2

    lax.fori_loop(0, _TA, tfma, 0)
    pltpu.sync_copy(hv0.at[tsl], acc.at[tidx.at[2]], add=True)

    plsc.subcore_barrier()

    @pl.when(s < 15)
    def _():
        pltpu.sync_copy(acc.at[pl.ds(r0, 624)], out.at[c, pl.ds(r0, 624)])

    @pl.when(s == 15)
    def _():
        pltpu.sync_copy(acc.at[pl.ds(9360, 640)], out.at[c, pl.ds(9360, 640)])


@functools.lru_cache(maxsize=None)
def _etype_kernel_build():
    mesh = plsc.VectorSubcoreMesh(core_axis_name="c", subcore_axis_name="s")
    return pl.kernel(
        _etype_body,
        out_type=jax.ShapeDtypeStruct((_E,), jnp.int32),
        mesh=mesh,
        scratch_types=[
            pltpu.VMEM((4, 2, _KE), jnp.int32),
            pltpu.VMEM((_KE,), jnp.int32),
            pltpu.VMEM((_KE,), jnp.int32),
            pltpu.VMEM((_KE,), jnp.int32),
            pltpu.VMEM((_KE,), jnp.int32),
            pltpu.VMEM((_KE,), jnp.int32),
            pltpu.VMEM((_KE,), jnp.int32),
            pltpu.VMEM((2, _TE), jnp.int32),
        ] + [pltpu.SemaphoreType.DMA] * 8,
    )


def _etype_call(src_i, dst_i, nt):
    return _etype_kernel_build()(src_i, dst_i, nt)


@functools.lru_cache(maxsize=None)
def _agg_kernel_build():
    mesh = plsc.VectorSubcoreMesh(core_axis_name="c", subcore_axis_name="s")
    return pl.kernel(
        _agg_body,
        out_type=jax.ShapeDtypeStruct((2, _N, _D), jnp.float32),
        mesh=mesh,
        scratch_types=[
            pltpu.VMEM((4, 3, _KA), jnp.int32),
            pltpu.VMEM((_KA, _D), jnp.float32),
            pltpu.VMEM((_KA, _D), jnp.float32),
            pltpu.VMEM((_KA, _D), jnp.float32),
            pltpu.VMEM((_KA, _D), jnp.float32),
            pltpu.VMEM((_KA, _D), jnp.float32),
            pltpu.VMEM((_KA, _D), jnp.float32),
            pltpu.VMEM((3, _TA), jnp.int32),
            pltpu.VMEM_SHARED((_N, _D), jnp.float32),
        ] + [pltpu.SemaphoreType.DMA] * 10,
    )


def _agg_call(src_i, et_i, dst_i, h, nn, tp):
    return _agg_kernel_build()(src_i, et_i, dst_i, h, nn, tp)


def _prep_call(nt2, atom, eemb, wlist):
    return pl.pallas_call(
        _prep_body,
        out_shape=[
            jax.ShapeDtypeStruct((_N, _D), jnp.float32),
            jax.ShapeDtypeStruct((_ETAB, _D), jnp.float32),
            jax.ShapeDtypeStruct((_ETAB, _D), jnp.float32),
            jax.ShapeDtypeStruct((_ETAB, _D), jnp.float32),
        ],
    )(nt2, atom, eemb, *wlist)


def _h_call(dist2, wlist):
    bcast = lambda shape: pl.BlockSpec(shape, lambda i: (0, 0))
    wspecs = []
    for w in wlist:
        wspecs.append(bcast(w.shape))
    return pl.pallas_call(
        _h_body,
        grid=(_GE,),
        in_specs=[pl.BlockSpec((_BE, 1), lambda i: (i, 0))] + wspecs,
        out_specs=[pl.BlockSpec((_BE, _D), lambda i: (i, 0))] * 3,
        out_shape=[jax.ShapeDtypeStruct((_E, _D), jnp.float32)] * 3,
    )(dist2, *wlist)


def _nn_call(node, w, b):
    return pl.pallas_call(
        _nn_body,
        grid=(_GN,),
        in_specs=[
            pl.BlockSpec((_BN, _D), lambda i: (i, 0)),
            pl.BlockSpec((_D, _D), lambda i: (0, 0)),
            pl.BlockSpec((1, _D), lambda i: (0, 0)),
        ],
        out_specs=pl.BlockSpec((_BN, _D), lambda i: (i, 0)),
        out_shape=jax.ShapeDtypeStruct((_N, _D), jnp.float32),
    )(node, w, b)


def _upd_call(parts, node, w2, b2, w3, b3):
    return pl.pallas_call(
        _upd_body,
        grid=(_GN,),
        in_specs=[
            pl.BlockSpec((2, _BN, _D), lambda i: (0, i, 0)),
            pl.BlockSpec((_BN, _D), lambda i: (i, 0)),
            pl.BlockSpec((_D, _D), lambda i: (0, 0)),
            pl.BlockSpec((1, _D), lambda i: (0, 0)),
            pl.BlockSpec((_D, _D), lambda i: (0, 0)),
            pl.BlockSpec((1, _D), lambda i: (0, 0)),
        ],
        out_specs=pl.BlockSpec((_BN, _D), lambda i: (i, 0)),
        out_shape=jax.ShapeDtypeStruct((_N, _D), jnp.float32),
    )(parts, node, w2, b2, w3, b3)


def _ro_call(nodes, d1ws, d1b, d2w, d2b):
    nspec = pl.BlockSpec((_BN, _D), lambda i: (i, 0))
    wspec = pl.BlockSpec((_D, 64), lambda i: (0, 0))
    return pl.pallas_call(
        _ro_body,
        grid=(_GN,),
        in_specs=[nspec] * 4 + [wspec] * 4 + [
            pl.BlockSpec((1, 64), lambda i: (0, 0)),
            pl.BlockSpec((64, 1), lambda i: (0, 0)),
            pl.BlockSpec((1, 1), lambda i: (0, 0)),
        ],
        out_specs=pl.BlockSpec((1, 1), lambda i: (0, 0)),
        out_shape=jax.ShapeDtypeStruct((1, 1), jnp.float32),
    )(*nodes, *d1ws, d1b, d2w, d2b)


def kernel(node_type, edge_index, distance, params):
    p = params
    nt = node_type.astype(jnp.int32)
    src = edge_index[0].astype(jnp.int32)
    dst = edge_index[1].astype(jnp.int32)
    dist2 = distance.astype(jnp.float32).reshape(_E, 1)
    convs = [p['conv_%d' % i] for i in range(3)]
    rb = lambda x: x.reshape(1, -1)

    prep_w = []
    for cv in convs:
        prep_w += [cv['ve3_w'], rb(cv['ve3_b']), cv['el1_w'], rb(cv['el1_b'])]
    node0, tp0, tp1, tp2 = _prep_call(nt.reshape(_N, 1), p['atom_emb'],
                                      p['edge_emb'], prep_w)

    h_w = []
    for cv in convs:
        h_w += [cv['ve1_w'], rb(cv['ve1_b']), cv['ve2_w'], rb(cv['ve2_b'])]
    hs = _h_call(dist2, h_w)

    etype = _etype_call(src, dst, nt)

    tps = [tp0, tp1, tp2]
    node = node0
    nodes = [node0]
    for i in range(3):
        cv = convs[i]
        nn = _nn_call(node, cv['nl1_w'], rb(cv['nl1_b']))
        parts = _agg_call(src, etype, dst, hs[i], nn, tps[i])
        node = _upd_call(parts, node, cv['nl2_w'], rb(cv['nl2_b']),
                         cv['nl3_w'], rb(cv['nl3_b']))
        nodes.append(node)

    d1ws = [p['d1_w'][i * _D:(i + 1) * _D] for i in range(4)]
    return _ro_call(nodes, d1ws, rb(p['d1_b']), p['d2_w'],
                    p['d2_b'].reshape(1, 1))
